# baseline (speedup 1.0000x reference)
import functools

import jax
import jax.numpy as jnp
import numpy as np
from jax import lax
from jax.experimental import pallas as pl
from jax.experimental.pallas import tpu as pltpu
from jax.experimental.pallas import tpu_sc as plsc

K = 20
NEG_SLOPE = 0.2
NC = 2
NS = 16
NW = NC * NS
LANES = 16
NG = 24


def _tc_block(xb_ref, rows_ref, w1t_ref, w21t_ref,
              nd_ref, thr_ref, y_ref, z_ref, *, k):
    xb = xb_ref[0]
    rows = rows_ref[0]
    r = rows.shape[0]

    g = lax.dot_general(rows, xb, (((1,), (1,)), ((), ())),
                        preferred_element_type=jnp.float32)
    xx_all = jnp.sum(xb * xb, axis=1)
    xx_rows = jnp.sum(rows * rows, axis=1)
    nd = 2.0 * g - xx_rows[:, None] - xx_all[None, :]
    nd_ref[0] = nd

    ibits = lax.bitcast_convert_type(nd, jnp.int32)
    key = jnp.where(ibits < 0, ibits ^ jnp.int32(0x7FFFFFFF), ibits)

    def bs_body(_, carry):
        lo, hi = carry
        mid = (lo >> 1) + (hi >> 1) + (lo & hi & 1)
        cnt = jnp.sum((key >= mid[:, None]).astype(jnp.int32), axis=1)
        ge = cnt >= k
        return jnp.where(ge, mid, lo), jnp.where(ge, hi, mid)

    lo0 = jnp.full((r,), np.int32(-2**31), jnp.int32)
    hi0 = jnp.full((r,), np.int32(2**31 - 1), jnp.int32)
    lo, _ = lax.fori_loop(0, 32, bs_body, (lo0, hi0))
    thr_bits = jnp.where(lo < 0, lo ^ jnp.int32(0x7FFFFFFF), lo)
    thr_ref[0, 0] = lax.bitcast_convert_type(thr_bits, jnp.float32)

    y_ref[0] = jnp.dot(rows, w1t_ref[...], preferred_element_type=jnp.float32)
    z_ref[0] = jnp.dot(rows, w21t_ref[...], preferred_element_type=jnp.float32)


def _tc_stage(x_t, w1t, w21t, *, rblk):
    b, n, c = x_t.shape
    o = w21t.shape[1]
    grid = (b, n // rblk)
    return pl.pallas_call(
        functools.partial(_tc_block, k=K),
        grid=grid,
        in_specs=[
            pl.BlockSpec((1, n, c), lambda bi, ri: (bi, 0, 0)),
            pl.BlockSpec((1, rblk, c), lambda bi, ri: (bi, ri, 0)),
            pl.BlockSpec((c, 2 * o), lambda bi, ri: (0, 0)),
            pl.BlockSpec((c, o), lambda bi, ri: (0, 0)),
        ],
        out_specs=[
            pl.BlockSpec((1, rblk, n), lambda bi, ri: (bi, ri, 0)),
            pl.BlockSpec((1, 1, rblk),
                         lambda bi, ri, _nb=n // rblk: (bi * _nb + ri, 0, 0)),
            pl.BlockSpec((1, rblk, 2 * o), lambda bi, ri: (bi, ri, 0)),
            pl.BlockSpec((1, rblk, o), lambda bi, ri: (bi, ri, 0)),
        ],
        out_shape=[
            jax.ShapeDtypeStruct((b, n, n), jnp.float32),
            jax.ShapeDtypeStruct((b * (n // rblk), 1, rblk), jnp.float32),
            jax.ShapeDtypeStruct((b, n, 2 * o), jnp.float32),
            jax.ShapeDtypeStruct((b, n, o), jnp.float32),
        ],
    )(x_t, x_t, w1t, w21t)


def _sc_stage(nd_flat, thr_flat, y_flat, z_flat, *, n, o, gb):
    total = nd_flat.shape[0]
    rpw = total // NW
    oq = o // LANES
    ngrp = gb * LANES

    mesh = plsc.VectorSubcoreMesh(core_axis_name="c", subcore_axis_name="s")

    @functools.partial(
        pl.kernel,
        out_type=jax.ShapeDtypeStruct((total, o), jnp.float32),
        mesh=mesh,
        compiler_params=pltpu.CompilerParams(needs_layout_passes=False),
        scratch_types=[
            pltpu.VMEM((gb, n), jnp.float32),
            pltpu.VMEM((gb, n), jnp.float32),
            pltpu.VMEM((rpw,), jnp.float32),
            pltpu.VMEM((rpw, NG), jnp.int32),
            pltpu.VMEM((NG, 2 * o), jnp.float32),
            pltpu.VMEM((NG, 2 * o), jnp.float32),
            pltpu.VMEM((rpw, o), jnp.float32),
            pltpu.SemaphoreType.DMA,
            pltpu.SemaphoreType.DMA,
            pltpu.SemaphoreType.DMA,
            pltpu.SemaphoreType.DMA,
        ],
    )
    def sc_kernel(nd_hbm, thr_hbm, y_hbm, z_hbm, out_hbm,
                  buf_a, buf_b, thr_v, idx_v, gat_a, gat_b, out_v,
                  sem_a, sem_b, sem_ga, sem_gb):
        wid = lax.axis_index("s") * NC + lax.axis_index("c")
        base = wid * rpw
        nbatch = rpw // gb

        pltpu.sync_copy(thr_hbm.at[pl.ds(base, rpw)], thr_v)
        pltpu.sync_copy(z_hbm.at[pl.ds(base, rpw)], out_v)

        iota = lax.broadcasted_iota(jnp.int32, (LANES,), 0)

        def row_copy(g, buf, sem):
            return pltpu.make_async_copy(
                nd_hbm.at[pl.ds(base + g * gb, gb)], buf, sem)

        def compress_batch(g, buf):
            def row_body(i, _):
                t = g * gb + i
                rglob = base + t
                ybase = (rglob // n) * n
                thr_b = plsc.load_gather(
                    thr_v, [jnp.full((LANES,), t, jnp.int32)])
                selfv = jnp.full((LANES,), rglob, jnp.int32)
                idx_v[t, pl.ds(0, LANES)] = selfv
                idx_v[t, pl.ds(NG - LANES, LANES)] = selfv

                def chunk_body(c, cnt):
                    v = buf[i, pl.ds(c * LANES, LANES)]
                    msk = v >= thr_b
                    npop = plsc.all_reduce_population_count(msk)

                    @pl.when(jnp.any(msk))
                    def _():
                        pos = cnt + plsc.cumsum(msk.astype(jnp.int32)) - 1
                        idxs = iota + (c * LANES + ybase)
                        rowsel = jnp.full((LANES,), t, jnp.int32)
                        plsc.store_scatter(idx_v, [rowsel, pos], idxs,
                                           mask=msk & (pos < NG))

                    return cnt + npop

                lax.fori_loop(0, n // LANES, chunk_body,
                              jnp.zeros((LANES,), jnp.int32))
                return 0

            lax.fori_loop(0, gb, row_body, 0)

        row_copy(0, buf_a, sem_a).start()
        def batch_loop(g, _):
            @pl.when(g % 2 == 0)
            def _():
                @pl.when(g + 1 < nbatch)
                def _():
                    row_copy(g + 1, buf_b, sem_b).start()
                row_copy(g, buf_a, sem_a).wait()
                compress_batch(g, buf_a)

            @pl.when(g % 2 == 1)
            def _():
                @pl.when(g + 1 < nbatch)
                def _():
                    row_copy(g + 1, buf_a, sem_a).start()
                row_copy(g, buf_b, sem_b).wait()
                compress_batch(g, buf_b)
            return 0

        lax.fori_loop(0, nbatch, batch_loop, 0)

        def gat_copy(t, gat, sem):
            return pltpu.make_async_copy(y_hbm.at[idx_v.at[t]], gat, sem)

        def max_phase(t, gat):
            acc = [gat[0, pl.ds(q * LANES, LANES)] for q in range(oq)]
            for j in range(1, NG):
                for q in range(oq):
                    acc[q] = jnp.maximum(acc[q],
                                         gat[j, pl.ds(q * LANES, LANES)])
            for q in range(oq):
                res = acc[q] + out_v[t, pl.ds(q * LANES, LANES)]
                out_v[t, pl.ds(q * LANES, LANES)] = jnp.maximum(
                    res, NEG_SLOPE * res)

        gat_copy(0, gat_a, sem_ga).start()
        def pair_loop(tt, _):
            t0 = 2 * tt
            gat_copy(t0 + 1, gat_b, sem_gb).start()
            gat_copy(t0, gat_a, sem_ga).wait()
            max_phase(t0, gat_a)

            @pl.when(t0 + 2 < rpw)
            def _():
                gat_copy(t0 + 2, gat_a, sem_ga).start()
            gat_copy(t0 + 1, gat_b, sem_gb).wait()
            max_phase(t0 + 1, gat_b)
            return 0

        lax.fori_loop(0, rpw // 2, pair_loop, 0)

        pltpu.sync_copy(out_v, out_hbm.at[pl.ds(base, rpw)])

    return sc_kernel(nd_flat, thr_flat, y_flat, z_flat)


@jax.jit
def kernel(x, W):
    b, c, n = x.shape
    o = W.shape[0]
    x_t = jnp.transpose(x, (0, 2, 1))
    w1 = W[:, :c]
    w2 = W[:, c:]
    w1t = jnp.transpose(w1)
    w21t = jnp.transpose(w2 - w1)
    w1t_pad = jnp.pad(w1t, ((0, 0), (0, o)))
    nd, thr, y, z = _tc_stage(x_t, w1t_pad, w21t, rblk=256)
    out_flat = _sc_stage(nd.reshape(b * n, n), thr.reshape(b * n),
                         y.reshape(b * n, 2 * o), z.reshape(b * n, o),
                         n=n, o=o, gb=8)
    return jnp.transpose(out_flat.reshape(b, n, o), (0, 2, 1))


# speedup vs baseline: 1.3229x; 1.3229x over previous
import functools

import jax
import jax.numpy as jnp
import numpy as np
from jax import lax
from jax.experimental import pallas as pl
from jax.experimental.pallas import tpu as pltpu
from jax.experimental.pallas import tpu_sc as plsc

K = 20
NEG_SLOPE = 0.2
NC = 2
NS = 16
NW = NC * NS
LANES = 16
NG = 24


def _tc_block(xb_ref, rows_ref, w1t_ref, w21t_ref,
              nd_ref, thr_ref, y_ref, z_ref, *, k):
    xb = xb_ref[0]
    rows = rows_ref[0]
    r = rows.shape[0]

    g = lax.dot_general(rows, xb, (((1,), (1,)), ((), ())),
                        preferred_element_type=jnp.float32)
    xx_all = jnp.sum(xb * xb, axis=1)
    xx_rows = jnp.sum(rows * rows, axis=1)
    nd = 2.0 * g - xx_rows[:, None] - xx_all[None, :]
    nd_ref[0] = nd

    ibits = lax.bitcast_convert_type(nd, jnp.int32)
    key = jnp.where(ibits < 0, ibits ^ jnp.int32(0x7FFFFFFF), ibits)

    def bs_body(_, carry):
        lo, hi = carry
        mid = (lo >> 1) + (hi >> 1) + (lo & hi & 1)
        cnt = jnp.sum((key >= mid[:, None]).astype(jnp.int32), axis=1)
        ge = cnt >= k
        return jnp.where(ge, mid, lo), jnp.where(ge, hi, mid)

    lo0 = jnp.full((r,), np.int32(-2**31), jnp.int32)
    hi0 = jnp.full((r,), np.int32(2**31 - 1), jnp.int32)
    lo, _ = lax.fori_loop(0, 32, bs_body, (lo0, hi0))
    thr_bits = jnp.where(lo < 0, lo ^ jnp.int32(0x7FFFFFFF), lo)
    thr_ref[0, 0] = lax.bitcast_convert_type(thr_bits, jnp.float32)

    y_ref[0] = jnp.dot(rows, w1t_ref[...], preferred_element_type=jnp.float32)
    z_ref[0] = jnp.dot(rows, w21t_ref[...], preferred_element_type=jnp.float32)


def _tc_stage(x_t, w1t, w21t, *, rblk):
    b, n, c = x_t.shape
    o = w21t.shape[1]
    grid = (b, n // rblk)
    return pl.pallas_call(
        functools.partial(_tc_block, k=K),
        grid=grid,
        in_specs=[
            pl.BlockSpec((1, n, c), lambda bi, ri: (bi, 0, 0)),
            pl.BlockSpec((1, rblk, c), lambda bi, ri: (bi, ri, 0)),
            pl.BlockSpec((c, 2 * o), lambda bi, ri: (0, 0)),
            pl.BlockSpec((c, o), lambda bi, ri: (0, 0)),
        ],
        out_specs=[
            pl.BlockSpec((1, rblk, n), lambda bi, ri: (bi, ri, 0)),
            pl.BlockSpec((1, 1, rblk),
                         lambda bi, ri, _nb=n // rblk: (bi * _nb + ri, 0, 0)),
            pl.BlockSpec((1, rblk, 2 * o), lambda bi, ri: (bi, ri, 0)),
            pl.BlockSpec((1, rblk, o), lambda bi, ri: (bi, ri, 0)),
        ],
        out_shape=[
            jax.ShapeDtypeStruct((b, n, n), jnp.float32),
            jax.ShapeDtypeStruct((b * (n // rblk), 1, rblk), jnp.float32),
            jax.ShapeDtypeStruct((b, n, 2 * o), jnp.float32),
            jax.ShapeDtypeStruct((b, n, o), jnp.float32),
        ],
    )(x_t, x_t, w1t, w21t)


def _sc_stage(nd_flat, thr_flat, y_flat, z_flat, *, n, o):
    total = nd_flat.shape[0]
    rpw = total // NW
    oq = o // LANES
    ngrp = rpw // LANES

    mesh = plsc.VectorSubcoreMesh(core_axis_name="c", subcore_axis_name="s")

    @functools.partial(
        pl.kernel,
        out_type=jax.ShapeDtypeStruct((total, o), jnp.float32),
        mesh=mesh,
        compiler_params=pltpu.CompilerParams(needs_layout_passes=False),
        scratch_types=[
            pltpu.VMEM((LANES, n), jnp.float32),
            pltpu.VMEM((LANES, n), jnp.float32),
            pltpu.VMEM((rpw,), jnp.float32),
            pltpu.VMEM((LANES, NG), jnp.int32),
            pltpu.VMEM((NG, 2 * o), jnp.float32),
            pltpu.VMEM((NG, 2 * o), jnp.float32),
            pltpu.VMEM((LANES, o), jnp.float32),
            pltpu.SemaphoreType.DMA,
            pltpu.SemaphoreType.DMA,
            pltpu.SemaphoreType.DMA,
            pltpu.SemaphoreType.DMA,
        ],
    )
    def sc_kernel(nd_hbm, thr_hbm, y_hbm, z_hbm, out_hbm,
                  buf_a, buf_b, thr_v, idx_v, gat_a, gat_b, out_v,
                  sem_a, sem_b, sem_ga, sem_gb):
        wid = lax.axis_index("s") * NC + lax.axis_index("c")
        base = wid * rpw
        ybase = (base // n) * n

        pltpu.sync_copy(thr_hbm.at[pl.ds(base, rpw)], thr_v)

        iota = lax.broadcasted_iota(jnp.int32, (LANES,), 0)
        ngv = jnp.full((LANES,), NG, jnp.int32)

        def nd_copy(g, buf, sem):
            return pltpu.make_async_copy(
                nd_hbm.at[pl.ds(base + g * LANES, LANES)], buf, sem)

        def compress_group(g, buf):
            t0 = g * LANES
            thr16 = thr_v[pl.ds(t0, LANES)]
            selfv = base + t0 + iota
            for sl in range(NG):
                plsc.store_scatter(idx_v, [iota, jnp.full((LANES,), sl,
                                                          jnp.int32)], selfv)

            def col_body(c, cnt):
                v = plsc.load_gather(buf, [iota, jnp.full((LANES,), c,
                                                          jnp.int32)])
                msk = v >= thr16
                ok = msk & (cnt < ngv)
                plsc.store_scatter(idx_v, [iota, cnt],
                                   jnp.full((LANES,), c + ybase, jnp.int32),
                                   mask=ok)
                return cnt + ok.astype(jnp.int32)

            lax.fori_loop(0, n, col_body, jnp.zeros((LANES,), jnp.int32),
                          unroll=4)

        def gat_copy(i, gat, sem):
            return pltpu.make_async_copy(y_hbm.at[idx_v.at[i]], gat, sem)

        def max_phase(i, gat):
            acc = [gat[0, pl.ds(q * LANES, LANES)] for q in range(oq)]
            for j in range(1, NG):
                for q in range(oq):
                    acc[q] = jnp.maximum(acc[q],
                                         gat[j, pl.ds(q * LANES, LANES)])
            for q in range(oq):
                res = acc[q] + out_v[i, pl.ds(q * LANES, LANES)]
                out_v[i, pl.ds(q * LANES, LANES)] = jnp.maximum(
                    res, NEG_SLOPE * res)

        def handle_group(g, buf):
            compress_group(g, buf)
            pltpu.sync_copy(z_hbm.at[pl.ds(base + g * LANES, LANES)], out_v)
            gat_copy(0, gat_a, sem_ga).start()

            def pair_loop(tt, _):
                i0 = 2 * tt
                gat_copy(i0 + 1, gat_b, sem_gb).start()
                gat_copy(i0, gat_a, sem_ga).wait()
                max_phase(i0, gat_a)

                @pl.when(i0 + 2 < LANES)
                def _():
                    gat_copy(i0 + 2, gat_a, sem_ga).start()
                gat_copy(i0 + 1, gat_b, sem_gb).wait()
                max_phase(i0 + 1, gat_b)
                return 0

            lax.fori_loop(0, LANES // 2, pair_loop, 0)
            pltpu.sync_copy(out_v, out_hbm.at[pl.ds(base + g * LANES, LANES)])

        nd_copy(0, buf_a, sem_a).start()

        def group_loop(g, _):
            @pl.when(g % 2 == 0)
            def _():
                @pl.when(g + 1 < ngrp)
                def _():
                    nd_copy(g + 1, buf_b, sem_b).start()
                nd_copy(g, buf_a, sem_a).wait()
                handle_group(g, buf_a)

            @pl.when(g % 2 == 1)
            def _():
                @pl.when(g + 1 < ngrp)
                def _():
                    nd_copy(g + 1, buf_a, sem_a).start()
                nd_copy(g, buf_b, sem_b).wait()
                handle_group(g, buf_b)
            return 0

        lax.fori_loop(0, ngrp, group_loop, 0)

    return sc_kernel(nd_flat, thr_flat, y_flat, z_flat)


@jax.jit
def kernel(x, W):
    b, c, n = x.shape
    o = W.shape[0]
    x_t = jnp.transpose(x, (0, 2, 1))
    w1 = W[:, :c]
    w2 = W[:, c:]
    w1t = jnp.transpose(w1)
    w21t = jnp.transpose(w2 - w1)
    w1t_pad = jnp.pad(w1t, ((0, 0), (0, o)))
    nd, thr, y, z = _tc_stage(x_t, w1t_pad, w21t, rblk=256)
    out_flat = _sc_stage(nd.reshape(b * n, n), thr.reshape(b * n),
                         y.reshape(b * n, 2 * o), z.reshape(b * n, o),
                         n=n, o=o)
    return jnp.transpose(out_flat.reshape(b, n, o), (0, 2, 1))


# speedup vs baseline: 2.0662x; 1.5619x over previous
import functools

import jax
import jax.numpy as jnp
import numpy as np
from jax import lax
from jax.experimental import pallas as pl
from jax.experimental.pallas import tpu as pltpu
from jax.experimental.pallas import tpu_sc as plsc

K = 20
NEG_SLOPE = 0.2
NC = 2
NS = 16
NW = NC * NS
LANES = 16
NG = 24
SLAB = 128
HALF = 128


def _tc_block(xb_ref, rows_ref, w1t_ref, w21t_ref,
              ndw_ref, thr_ref, y_ref, z_ref, *, k):
    xb = xb_ref[0]
    rows = rows_ref[0]
    r = rows.shape[0]

    gt = lax.dot_general(xb, rows, (((1,), (1,)), ((), ())),
                         preferred_element_type=jnp.float32)
    xx_all = jnp.sum(xb * xb, axis=1)
    xx_rows = jnp.sum(rows * rows, axis=1)
    ndt = 2.0 * gt - xx_all[:, None] - xx_rows[None, :]
    ndw_ref[0, 0] = ndt

    ibits = lax.bitcast_convert_type(ndt, jnp.int32)
    key = jnp.where(ibits < 0, ibits ^ jnp.int32(0x7FFFFFFF), ibits)

    def bs_body(_, carry):
        lo, hi = carry
        mid = (lo >> 1) + (hi >> 1) + (lo & hi & 1)
        cnt = jnp.sum((key >= mid[None, :]).astype(jnp.int32), axis=0)
        ge = cnt >= k
        return jnp.where(ge, mid, lo), jnp.where(ge, hi, mid)

    lo0 = jnp.full((r,), np.int32(-2**31), jnp.int32)
    hi0 = jnp.full((r,), np.int32(2**31 - 1), jnp.int32)
    lo, _ = lax.fori_loop(0, 32, bs_body, (lo0, hi0))
    thr_bits = jnp.where(lo < 0, lo ^ jnp.int32(0x7FFFFFFF), lo)
    thr_ref[0, 0] = lax.bitcast_convert_type(thr_bits, jnp.float32)

    y_ref[0] = jnp.dot(rows, w1t_ref[...], preferred_element_type=jnp.float32)
    z_ref[0] = jnp.dot(rows, w21t_ref[...], preferred_element_type=jnp.float32)


def _tc_stage(x_t, w1t, w21t, *, rblk):
    b, n, c = x_t.shape
    o = w21t.shape[1]
    nblk = n // rblk
    grid = (b, nblk)
    return pl.pallas_call(
        functools.partial(_tc_block, k=K),
        grid=grid,
        in_specs=[
            pl.BlockSpec((1, n, c), lambda bi, ri: (bi, 0, 0)),
            pl.BlockSpec((1, rblk, c), lambda bi, ri: (bi, ri, 0)),
            pl.BlockSpec((c, 2 * o), lambda bi, ri: (0, 0)),
            pl.BlockSpec((c, o), lambda bi, ri: (0, 0)),
        ],
        out_specs=[
            pl.BlockSpec((1, 1, n, rblk), lambda bi, ri: (bi, ri, 0, 0)),
            pl.BlockSpec((1, 1, rblk),
                         lambda bi, ri, _nb=nblk: (bi * _nb + ri, 0, 0)),
            pl.BlockSpec((1, rblk, 2 * o), lambda bi, ri: (bi, ri, 0)),
            pl.BlockSpec((1, rblk, o), lambda bi, ri: (bi, ri, 0)),
        ],
        out_shape=[
            jax.ShapeDtypeStruct((b, nblk, n, rblk), jnp.float32),
            jax.ShapeDtypeStruct((b * nblk, 1, rblk), jnp.float32),
            jax.ShapeDtypeStruct((b, n, 2 * o), jnp.float32),
            jax.ShapeDtypeStruct((b, n, o), jnp.float32),
        ],
    )(x_t, x_t, w1t, w21t)


def _sc_stage(ndw, thr_flat, y_flat, z_flat, *, n, o):
    total = thr_flat.shape[0]
    rpw = total // NW
    nblk = ndw.shape[1]
    oq = o // LANES
    nslab = n // SLAB
    nhalf = rpw // HALF
    nu = nslab * nhalf
    grp = HALF // LANES
    qrows = HALF // 2

    mesh = plsc.VectorSubcoreMesh(core_axis_name="c", subcore_axis_name="s")

    @functools.partial(
        pl.kernel,
        out_type=jax.ShapeDtypeStruct((total, o), jnp.float32),
        mesh=mesh,
        compiler_params=pltpu.CompilerParams(needs_layout_passes=False),
        scratch_types=[
            pltpu.VMEM((SLAB, HALF), jnp.float32),
            pltpu.VMEM((SLAB, HALF), jnp.float32),
            pltpu.VMEM((rpw,), jnp.float32),
            pltpu.VMEM((HALF, NG), jnp.int32),
            pltpu.VMEM((NG, 2 * o), jnp.float32),
            pltpu.VMEM((NG, 2 * o), jnp.float32),
            pltpu.VMEM((HALF // 2, o), jnp.float32),
            pltpu.VMEM((HALF // LANES, LANES), jnp.int32),
            pltpu.SemaphoreType.DMA,
            pltpu.SemaphoreType.DMA,
            pltpu.SemaphoreType.DMA,
            pltpu.SemaphoreType.DMA,
        ],
    )
    def sc_kernel(ndw_hbm, thr_hbm, y_hbm, z_hbm, out_hbm,
                  buf_a, buf_b, thr_v, idx_v, gat_a, gat_b, out_v, cnt_v,
                  sem_a, sem_b, sem_ga, sem_gb):
        wid = lax.axis_index("s") * NC + lax.axis_index("c")
        base = wid * rpw
        bi = wid // nblk
        ri = wid % nblk
        ybase = bi * n

        pltpu.sync_copy(thr_hbm.at[pl.ds(base, rpw)], thr_v)

        iota = lax.broadcasted_iota(jnp.int32, (LANES,), 0)
        ngv = jnp.full((LANES,), NG, jnp.int32)
        zeros16 = jnp.zeros((LANES,), jnp.int32)

        def slab_copy(u, buf, sem):
            return pltpu.make_async_copy(
                ndw_hbm.at[bi, ri,
                           pl.ds((u % nslab) * SLAB, SLAB),
                           pl.ds((u // nslab) * HALF, HALF)],
                buf, sem)

        def process_slab(u, buf):
            jb = ybase + (u % nslab) * SLAB
            h0 = (u // nslab) * HALF
            for kg in range(grp):
                thr16 = thr_v[pl.ds(h0 + kg * LANES, LANES)]
                rowv = iota + (kg * LANES)

                def colb(c, cnt):
                    v = buf[c, pl.ds(kg * LANES, LANES)]
                    msk = v >= thr16
                    ok = msk & (cnt < ngv)
                    val = jnp.full((LANES,), jb + c, jnp.int32)
                    plsc.store_scatter(idx_v, [rowv, cnt], val, mask=ok)
                    return cnt + ok.astype(jnp.int32)

                cnt1 = lax.fori_loop(0, SLAB, colb,
                                     cnt_v[kg, pl.ds(0, LANES)], unroll=4)
                cnt_v[kg, pl.ds(0, LANES)] = cnt1

        def uloop(u, _):
            @pl.when(u % 2 == 0)
            def _():
                @pl.when(u + 1 < nu)
                def _():
                    slab_copy(u + 1, buf_b, sem_b).start()
                slab_copy(u, buf_a, sem_a).wait()
                process_slab(u, buf_a)

            @pl.when(u % 2 == 1)
            def _():
                @pl.when(u + 1 < nu)
                def _():
                    slab_copy(u + 1, buf_a, sem_a).start()
                slab_copy(u, buf_b, sem_b).wait()
                process_slab(u, buf_b)
            return 0

        def half_init(h):
            for kg in range(grp):
                cnt_v[kg, pl.ds(0, LANES)] = zeros16
                selfv = base + h * HALF + kg * LANES + iota
                rowv = iota + (kg * LANES)
                for sl in range(NG):
                    plsc.store_scatter(
                        idx_v, [rowv, jnp.full((LANES,), sl, jnp.int32)],
                        selfv)

        def gat_copy(i, gat, sem):
            return pltpu.make_async_copy(y_hbm.at[idx_v.at[i]], gat, sem)

        def max_phase(i_out, gat):
            acc = [gat[0, pl.ds(q * LANES, LANES)] for q in range(oq)]
            for j in range(1, NG):
                for q in range(oq):
                    acc[q] = jnp.maximum(acc[q],
                                         gat[j, pl.ds(q * LANES, LANES)])
            for q in range(oq):
                res = acc[q] + out_v[i_out, pl.ds(q * LANES, LANES)]
                out_v[i_out, pl.ds(q * LANES, LANES)] = jnp.maximum(
                    res, NEG_SLOPE * res)

        def gather_half(h):
            qrows_ = HALF // 2
            for q2 in range(2):
                r0 = h * HALF + q2 * qrows_
                pltpu.sync_copy(z_hbm.at[pl.ds(base + r0, qrows_)], out_v)
                gat_copy(q2 * qrows_, gat_a, sem_ga).start()

                def pair_loop(tt, _):
                    il = 2 * tt
                    i0 = q2 * qrows_ + il
                    gat_copy(i0 + 1, gat_b, sem_gb).start()
                    gat_copy(i0, gat_a, sem_ga).wait()
                    max_phase(il, gat_a)

                    @pl.when(il + 2 < qrows_)
                    def _():
                        gat_copy(i0 + 2, gat_a, sem_ga).start()
                    gat_copy(i0 + 1, gat_b, sem_gb).wait()
                    max_phase(il + 1, gat_b)
                    return 0

                lax.fori_loop(0, qrows_ // 2, pair_loop, 0)
                pltpu.sync_copy(out_v,
                                out_hbm.at[pl.ds(base + r0, qrows_)])

        slab_copy(0, buf_a, sem_a).start()
        half_init(0)
        lax.fori_loop(0, nslab, uloop, 0)
        gather_half(0)
        half_init(1)
        lax.fori_loop(nslab, nu, uloop, 0)
        gather_half(1)

    return sc_kernel(ndw, thr_flat, y_flat, z_flat)


@jax.jit
def kernel(x, W):
    b, c, n = x.shape
    o = W.shape[0]
    x_t = jnp.transpose(x, (0, 2, 1))
    w1 = W[:, :c]
    w2 = W[:, c:]
    w1t = jnp.transpose(w1)
    w21t = jnp.transpose(w2 - w1)
    w1t_pad = jnp.pad(w1t, ((0, 0), (0, o)))
    ndw, thr, y, z = _tc_stage(x_t, w1t_pad, w21t, rblk=256)
    out_flat = _sc_stage(ndw, thr.reshape(b * n),
                         y.reshape(b * n, 2 * o), z.reshape(b * n, o),
                         n=n, o=o)
    return jnp.transpose(out_flat.reshape(b, n, o), (0, 2, 1))


# speedup vs baseline: 2.2722x; 1.0997x over previous
import functools

import jax
import jax.numpy as jnp
import numpy as np
from jax import lax
from jax.experimental import pallas as pl
from jax.experimental.pallas import tpu as pltpu
from jax.experimental.pallas import tpu_sc as plsc

K = 20
NEG_SLOPE = 0.2
NC = 2
NS = 16
NW = NC * NS
LANES = 16
NG = 20
SLAB = 128
HALF = 128


def _tc_block(xb_ref, rows_ref, w1t_ref, w21t_ref,
              ndw_ref, thr_ref, y_ref, z_ref, *, k):
    xb = xb_ref[0]
    rows = rows_ref[0]
    r = rows.shape[0]

    gt = lax.dot_general(xb, rows, (((1,), (1,)), ((), ())),
                         preferred_element_type=jnp.float32)
    xx_all = jnp.sum(xb * xb, axis=1)
    xx_rows = jnp.sum(rows * rows, axis=1)
    ndt = 2.0 * gt - xx_all[:, None] - xx_rows[None, :]
    ndw_ref[0, 0] = ndt

    ibits = lax.bitcast_convert_type(ndt, jnp.int32)
    key = jnp.where(ibits < 0, ibits ^ jnp.int32(0x7FFFFFFF), ibits)

    def bs_body(_, carry):
        lo, hi = carry
        mid = (lo >> 1) + (hi >> 1) + (lo & hi & 1)
        cnt = jnp.sum((key >= mid[None, :]).astype(jnp.int32), axis=0)
        ge = cnt >= k
        return jnp.where(ge, mid, lo), jnp.where(ge, hi, mid)

    lo0 = jnp.full((r,), np.int32(-2**31), jnp.int32)
    hi0 = jnp.full((r,), np.int32(2**31 - 1), jnp.int32)
    lo, _ = lax.fori_loop(0, 32, bs_body, (lo0, hi0))
    thr_bits = jnp.where(lo < 0, lo ^ jnp.int32(0x7FFFFFFF), lo)
    thr_ref[0, 0] = lax.bitcast_convert_type(thr_bits, jnp.float32)

    y_ref[0] = jnp.dot(rows, w1t_ref[...], preferred_element_type=jnp.float32)
    z_ref[0] = jnp.dot(rows, w21t_ref[...], preferred_element_type=jnp.float32)


def _tc_stage(x_t, w1t, w21t, *, rblk):
    b, n, c = x_t.shape
    o = w21t.shape[1]
    nblk = n // rblk
    grid = (b, nblk)
    return pl.pallas_call(
        functools.partial(_tc_block, k=K),
        grid=grid,
        in_specs=[
            pl.BlockSpec((1, n, c), lambda bi, ri: (bi, 0, 0)),
            pl.BlockSpec((1, rblk, c), lambda bi, ri: (bi, ri, 0)),
            pl.BlockSpec((c, 2 * o), lambda bi, ri: (0, 0)),
            pl.BlockSpec((c, o), lambda bi, ri: (0, 0)),
        ],
        out_specs=[
            pl.BlockSpec((1, 1, n, rblk), lambda bi, ri: (bi, ri, 0, 0)),
            pl.BlockSpec((1, 1, rblk),
                         lambda bi, ri, _nb=nblk: (bi * _nb + ri, 0, 0)),
            pl.BlockSpec((1, rblk, 2 * o), lambda bi, ri: (bi, ri, 0)),
            pl.BlockSpec((1, rblk, o), lambda bi, ri: (bi, ri, 0)),
        ],
        out_shape=[
            jax.ShapeDtypeStruct((b, nblk, n, rblk), jnp.float32),
            jax.ShapeDtypeStruct((b * nblk, 1, rblk), jnp.float32),
            jax.ShapeDtypeStruct((b, n, 2 * o), jnp.float32),
            jax.ShapeDtypeStruct((b, n, o), jnp.float32),
        ],
    )(x_t, x_t, w1t, w21t)


def _sc_stage(ndw, thr_flat, y_flat, z_flat, *, n, o):
    total = thr_flat.shape[0]
    rpw = total // NW
    nblk = ndw.shape[1]
    oq = o // LANES
    nslab = n // SLAB
    nhalf = rpw // HALF
    nu = nslab * nhalf
    grp = HALF // LANES
    qrows = HALF // 2

    mesh = plsc.VectorSubcoreMesh(core_axis_name="c", subcore_axis_name="s")

    @functools.partial(
        pl.kernel,
        out_type=jax.ShapeDtypeStruct((total, o), jnp.float32),
        mesh=mesh,
        compiler_params=pltpu.CompilerParams(needs_layout_passes=False),
        scratch_types=[
            pltpu.VMEM((SLAB, HALF), jnp.float32),
            pltpu.VMEM((SLAB, HALF), jnp.float32),
            pltpu.VMEM((rpw,), jnp.float32),
            pltpu.VMEM((HALF * NG,), jnp.int32),
            pltpu.VMEM((4 * NG, 2 * o), jnp.float32),
            pltpu.VMEM((4 * NG, 2 * o), jnp.float32),
            pltpu.VMEM((HALF // 2, o), jnp.float32),
            pltpu.VMEM((HALF // LANES, LANES), jnp.int32),
            pltpu.SemaphoreType.DMA,
            pltpu.SemaphoreType.DMA,
            pltpu.SemaphoreType.DMA,
            pltpu.SemaphoreType.DMA,
        ],
    )
    def sc_kernel(ndw_hbm, thr_hbm, y_hbm, z_hbm, out_hbm,
                  buf_a, buf_b, thr_v, idx_v, gat_a, gat_b, out_v, cnt_v,
                  sem_a, sem_b, sem_ga, sem_gb):
        wid = lax.axis_index("s") * NC + lax.axis_index("c")
        base = wid * rpw
        bi = wid // nblk
        ri = wid % nblk
        ybase = bi * n

        pltpu.sync_copy(thr_hbm.at[pl.ds(base, rpw)], thr_v)

        iota = lax.broadcasted_iota(jnp.int32, (LANES,), 0)
        ngv = jnp.full((LANES,), NG, jnp.int32)
        zeros16 = jnp.zeros((LANES,), jnp.int32)

        def slab_copy(u, buf, sem):
            return pltpu.make_async_copy(
                ndw_hbm.at[bi, ri,
                           pl.ds((u % nslab) * SLAB, SLAB),
                           pl.ds((u // nslab) * HALF, HALF)],
                buf, sem)

        def process_slab(u, buf):
            jb = ybase + (u % nslab) * SLAB
            h0 = (u // nslab) * HALF
            for kg in range(grp):
                thr16 = thr_v[pl.ds(h0 + kg * LANES, LANES)]
                posb = (iota + kg * LANES) * NG

                def colb(c, cnt):
                    v = buf[c, pl.ds(kg * LANES, LANES)]
                    msk = v >= thr16
                    ok = msk & (cnt < ngv)
                    val = jnp.full((LANES,), jb + c, jnp.int32)
                    plsc.store_scatter(idx_v, [posb + cnt], val, mask=ok)
                    return cnt + ok.astype(jnp.int32)

                cnt1 = lax.fori_loop(0, SLAB, colb,
                                     cnt_v[kg, pl.ds(0, LANES)], unroll=4)
                cnt_v[kg, pl.ds(0, LANES)] = cnt1

        def uloop(u, _):
            @pl.when(u % 2 == 0)
            def _():
                @pl.when(u + 1 < nu)
                def _():
                    slab_copy(u + 1, buf_b, sem_b).start()
                slab_copy(u, buf_a, sem_a).wait()
                process_slab(u, buf_a)

            @pl.when(u % 2 == 1)
            def _():
                @pl.when(u + 1 < nu)
                def _():
                    slab_copy(u + 1, buf_a, sem_a).start()
                slab_copy(u, buf_b, sem_b).wait()
                process_slab(u, buf_b)
            return 0

        def half_init(h):
            for kg in range(grp):
                cnt_v[kg, pl.ds(0, LANES)] = zeros16
                selfv = base + h * HALF + kg * LANES + iota
                posb = (iota + kg * LANES) * NG
                for sl in range(NG):
                    plsc.store_scatter(idx_v, [posb + sl], selfv)

        def gat_copy(i, gat, sem):
            return pltpu.make_async_copy(
                y_hbm.at[idx_v.at[pl.ds(i * NG, 4 * NG)]], gat, sem)

        def max_phase(jj, i_out, gat):
            acc = [gat[jj * NG, pl.ds(q * LANES, LANES)] for q in range(oq)]
            for j in range(1, NG):
                for q in range(oq):
                    acc[q] = jnp.maximum(
                        acc[q], gat[jj * NG + j, pl.ds(q * LANES, LANES)])
            for q in range(oq):
                res = acc[q] + out_v[i_out, pl.ds(q * LANES, LANES)]
                out_v[i_out, pl.ds(q * LANES, LANES)] = jnp.maximum(
                    res, NEG_SLOPE * res)

        def gather_half(h):
            qrows_ = HALF // 2
            for q2 in range(2):
                r0 = h * HALF + q2 * qrows_
                pltpu.sync_copy(z_hbm.at[pl.ds(base + r0, qrows_)], out_v)
                gat_copy(q2 * qrows_, gat_a, sem_ga).start()

                def pair_loop(tt, _):
                    il = 8 * tt
                    i0 = q2 * qrows_ + il
                    gat_copy(i0 + 4, gat_b, sem_gb).start()
                    gat_copy(i0, gat_a, sem_ga).wait()
                    for jj in range(4):
                        max_phase(jj, il + jj, gat_a)

                    @pl.when(il + 8 < qrows_)
                    def _():
                        gat_copy(i0 + 8, gat_a, sem_ga).start()
                    gat_copy(i0 + 4, gat_b, sem_gb).wait()
                    for jj in range(4):
                        max_phase(jj, il + 4 + jj, gat_b)
                    return 0

                lax.fori_loop(0, qrows_ // 8, pair_loop, 0)
                pltpu.sync_copy(out_v,
                                out_hbm.at[pl.ds(base + r0, qrows_)])

        slab_copy(0, buf_a, sem_a).start()
        half_init(0)
        lax.fori_loop(0, nslab, uloop, 0)
        gather_half(0)
        half_init(1)
        lax.fori_loop(nslab, nu, uloop, 0)
        gather_half(1)

    return sc_kernel(ndw, thr_flat, y_flat, z_flat)


@jax.jit
def kernel(x, W):
    b, c, n = x.shape
    o = W.shape[0]
    x_t = jnp.transpose(x, (0, 2, 1))
    w1 = W[:, :c]
    w2 = W[:, c:]
    w1t = jnp.transpose(w1)
    w21t = jnp.transpose(w2 - w1)
    w1t_pad = jnp.pad(w1t, ((0, 0), (0, o)))
    ndw, thr, y, z = _tc_stage(x_t, w1t_pad, w21t, rblk=256)
    out_flat = _sc_stage(ndw, thr.reshape(b * n),
                         y.reshape(b * n, 2 * o), z.reshape(b * n, o),
                         n=n, o=o)
    return jnp.transpose(out_flat.reshape(b, n, o), (0, 2, 1))


# speedup vs baseline: 2.3338x; 1.0271x over previous
import functools

import jax
import jax.numpy as jnp
import numpy as np
from jax import lax
from jax.experimental import pallas as pl
from jax.experimental.pallas import tpu as pltpu
from jax.experimental.pallas import tpu_sc as plsc

K = 20
NEG_SLOPE = 0.2
NC = 2
NS = 16
NW = NC * NS
LANES = 16
NG = 20
SLAB = 128
HALF = 64


def _tc_block(xb_ref, rows_ref, w1t_ref, w21t_ref,
              ndw_ref, thr_ref, y_ref, z_ref, *, k):
    xb = xb_ref[0]
    rows = rows_ref[0]
    r = rows.shape[0]

    gt = lax.dot_general(xb, rows, (((1,), (1,)), ((), ())),
                         preferred_element_type=jnp.float32)
    xx_all = jnp.sum(xb * xb, axis=1)
    xx_rows = jnp.sum(rows * rows, axis=1)
    ndt = 2.0 * gt - xx_all[:, None] - xx_rows[None, :]
    ndw_ref[0, 0] = ndt

    ibits = lax.bitcast_convert_type(ndt, jnp.int32)
    key = jnp.where(ibits < 0, ibits ^ jnp.int32(0x7FFFFFFF), ibits)

    def bs_body(_, carry):
        lo, hi = carry
        mid = (lo >> 1) + (hi >> 1) + (lo & hi & 1)
        cnt = jnp.sum((key >= mid[None, :]).astype(jnp.int32), axis=0)
        ge = cnt >= k
        return jnp.where(ge, mid, lo), jnp.where(ge, hi, mid)

    lo0 = jnp.full((r,), np.int32(-2**31), jnp.int32)
    hi0 = jnp.full((r,), np.int32(2**31 - 1), jnp.int32)
    lo, _ = lax.fori_loop(0, 32, bs_body, (lo0, hi0))
    thr_bits = jnp.where(lo < 0, lo ^ jnp.int32(0x7FFFFFFF), lo)
    thr_ref[0, 0] = lax.bitcast_convert_type(thr_bits, jnp.float32)

    y_ref[0] = jnp.dot(rows, w1t_ref[...], preferred_element_type=jnp.float32)
    z_ref[0] = jnp.dot(rows, w21t_ref[...], preferred_element_type=jnp.float32)


def _tc_stage(x_t, w1t, w21t, *, rblk):
    b, n, c = x_t.shape
    o = w21t.shape[1]
    nblk = n // rblk
    grid = (b, nblk)
    return pl.pallas_call(
        functools.partial(_tc_block, k=K),
        grid=grid,
        in_specs=[
            pl.BlockSpec((1, n, c), lambda bi, ri: (bi, 0, 0)),
            pl.BlockSpec((1, rblk, c), lambda bi, ri: (bi, ri, 0)),
            pl.BlockSpec((c, 2 * o), lambda bi, ri: (0, 0)),
            pl.BlockSpec((c, o), lambda bi, ri: (0, 0)),
        ],
        out_specs=[
            pl.BlockSpec((1, 1, n, rblk), lambda bi, ri: (bi, ri, 0, 0)),
            pl.BlockSpec((1, 1, rblk),
                         lambda bi, ri, _nb=nblk: (bi * _nb + ri, 0, 0)),
            pl.BlockSpec((1, rblk, 2 * o), lambda bi, ri: (bi, ri, 0)),
            pl.BlockSpec((1, rblk, o), lambda bi, ri: (bi, ri, 0)),
        ],
        out_shape=[
            jax.ShapeDtypeStruct((b, nblk, n, rblk), jnp.float32),
            jax.ShapeDtypeStruct((b * nblk, 1, rblk), jnp.float32),
            jax.ShapeDtypeStruct((b, n, 2 * o), jnp.float32),
            jax.ShapeDtypeStruct((b, n, o), jnp.float32),
        ],
    )(x_t, x_t, w1t, w21t)


def _sc_stage(ndw, thr_flat, y_flat, z_flat, *, n, o):
    total = thr_flat.shape[0]
    rpw = total // NW
    nblk = ndw.shape[1]
    rblk = n // nblk
    oq = o // LANES
    nslab = n // SLAB
    nhalf = rpw // HALF
    nu = nslab * nhalf
    grp = HALF // LANES
    qrows = HALF // 2

    mesh = plsc.VectorSubcoreMesh(core_axis_name="c", subcore_axis_name="s")

    @functools.partial(
        pl.kernel,
        out_type=jax.ShapeDtypeStruct((total, o), jnp.float32),
        mesh=mesh,
        compiler_params=pltpu.CompilerParams(needs_layout_passes=False),
        scratch_types=[
            pltpu.VMEM((SLAB, 128), jnp.float32),
            pltpu.VMEM((SLAB, 128), jnp.float32),
            pltpu.VMEM((rpw,), jnp.float32),
            pltpu.VMEM((HALF * NG,), jnp.int32),
            pltpu.VMEM((4 * NG, 2 * o), jnp.float32),
            pltpu.VMEM((4 * NG, 2 * o), jnp.float32),
            pltpu.VMEM((HALF // 2, o), jnp.float32),
            pltpu.VMEM((HALF // LANES, LANES), jnp.int32),
            pltpu.SemaphoreType.DMA,
            pltpu.SemaphoreType.DMA,
            pltpu.SemaphoreType.DMA,
            pltpu.SemaphoreType.DMA,
        ],
    )
    def sc_kernel(ndw_hbm, thr_hbm, y_hbm, z_hbm, out_hbm,
                  buf_a, buf_b, thr_v, idx_v, gat_a, gat_b, out_v, cnt_v,
                  sem_a, sem_b, sem_ga, sem_gb):
        wid = lax.axis_index("s") * NC + lax.axis_index("c")
        base = wid * rpw
        ri = base // rblk
        coff = base % rblk
        cal = (coff // 128) * 128
        boff = coff % 128
        ybase = 0

        pltpu.sync_copy(thr_hbm.at[pl.ds(base, rpw)], thr_v)

        iota = lax.broadcasted_iota(jnp.int32, (LANES,), 0)
        ngv = jnp.full((LANES,), NG, jnp.int32)
        zeros16 = jnp.zeros((LANES,), jnp.int32)

        def slab_copy(u, buf, sem):
            return pltpu.make_async_copy(
                ndw_hbm.at[0, ri,
                           pl.ds((u % nslab) * SLAB, SLAB),
                           pl.ds(cal, 128)],
                buf, sem)

        def process_slab(u, buf):
            jb = ybase + (u % nslab) * SLAB
            h0 = (u // nslab) * HALF
            for kg in range(grp):
                thr16 = thr_v[pl.ds(h0 + kg * LANES, LANES)]
                posb = (iota + kg * LANES) * NG

                def colb(c, cnt):
                    v = buf[c, pl.ds(boff + kg * LANES, LANES)]
                    msk = v >= thr16
                    ok = msk & (cnt < ngv)
                    val = jnp.full((LANES,), jb + c, jnp.int32)
                    plsc.store_scatter(idx_v, [posb + cnt], val, mask=ok)
                    return cnt + ok.astype(jnp.int32)

                cnt1 = lax.fori_loop(0, SLAB, colb,
                                     cnt_v[kg, pl.ds(0, LANES)], unroll=4)
                cnt_v[kg, pl.ds(0, LANES)] = cnt1

        def uloop(u, _):
            @pl.when(u % 2 == 0)
            def _():
                @pl.when(u + 1 < nu)
                def _():
                    slab_copy(u + 1, buf_b, sem_b).start()
                slab_copy(u, buf_a, sem_a).wait()
                process_slab(u, buf_a)

            @pl.when(u % 2 == 1)
            def _():
                @pl.when(u + 1 < nu)
                def _():
                    slab_copy(u + 1, buf_a, sem_a).start()
                slab_copy(u, buf_b, sem_b).wait()
                process_slab(u, buf_b)
            return 0

        def half_init(h):
            for kg in range(grp):
                cnt_v[kg, pl.ds(0, LANES)] = zeros16
                selfv = base + h * HALF + kg * LANES + iota
                posb = (iota + kg * LANES) * NG
                for sl in range(NG):
                    plsc.store_scatter(idx_v, [posb + sl], selfv)

        def gat_copy(i, gat, sem):
            return pltpu.make_async_copy(
                y_hbm.at[idx_v.at[pl.ds(i * NG, 4 * NG)]], gat, sem)

        def max_phase(jj, i_out, gat):
            acc = [gat[jj * NG, pl.ds(q * LANES, LANES)] for q in range(oq)]
            for j in range(1, NG):
                for q in range(oq):
                    acc[q] = jnp.maximum(
                        acc[q], gat[jj * NG + j, pl.ds(q * LANES, LANES)])
            for q in range(oq):
                res = acc[q] + out_v[i_out, pl.ds(q * LANES, LANES)]
                out_v[i_out, pl.ds(q * LANES, LANES)] = jnp.maximum(
                    res, NEG_SLOPE * res)

        def gather_half(h):
            qrows_ = HALF // 2
            for q2 in range(2):
                r0 = h * HALF + q2 * qrows_
                pltpu.sync_copy(z_hbm.at[pl.ds(base + r0, qrows_)], out_v)
                gat_copy(q2 * qrows_, gat_a, sem_ga).start()

                def pair_loop(tt, _):
                    il = 8 * tt
                    i0 = q2 * qrows_ + il
                    gat_copy(i0 + 4, gat_b, sem_gb).start()
                    gat_copy(i0, gat_a, sem_ga).wait()
                    for jj in range(4):
                        max_phase(jj, il + jj, gat_a)

                    @pl.when(il + 8 < qrows_)
                    def _():
                        gat_copy(i0 + 8, gat_a, sem_ga).start()
                    gat_copy(i0 + 4, gat_b, sem_gb).wait()
                    for jj in range(4):
                        max_phase(jj, il + 4 + jj, gat_b)
                    return 0

                lax.fori_loop(0, qrows_ // 8, pair_loop, 0)
                pltpu.sync_copy(out_v,
                                out_hbm.at[pl.ds(base + r0, qrows_)])

        slab_copy(0, buf_a, sem_a).start()
        half_init(0)
        lax.fori_loop(0, nslab, uloop, 0)
        gather_half(0)
        half_init(1)
        lax.fori_loop(nslab, nu, uloop, 0)
        gather_half(1)

    return sc_kernel(ndw, thr_flat, y_flat, z_flat)


@jax.jit
def kernel(x, W):
    b, c, n = x.shape
    o = W.shape[0]
    x_t = jnp.transpose(x, (0, 2, 1))
    w1 = W[:, :c]
    w2 = W[:, c:]
    w1t = jnp.transpose(w1)
    w21t = jnp.transpose(w2 - w1)
    w1t_pad = jnp.pad(w1t, ((0, 0), (0, o)))
    outs = []
    for bb in range(b):
        ndw, thr, y, z = _tc_stage(x_t[bb:bb + 1], w1t_pad, w21t, rblk=256)
        outs.append(_sc_stage(ndw, thr.reshape(n),
                              y.reshape(n, 2 * o), z.reshape(n, o),
                              n=n, o=o))
    return jnp.transpose(jnp.stack(outs), (0, 2, 1))


# speedup vs baseline: 2.9524x; 1.2651x over previous
import functools

import jax
import jax.numpy as jnp
import numpy as np
from jax import lax
from jax.experimental import pallas as pl
from jax.experimental.pallas import tpu as pltpu
from jax.experimental.pallas import tpu_sc as plsc

K = 20
NEG_SLOPE = 0.2
NC = 2
NS = 16
NW = NC * NS
LANES = 16
NG = 20
SLAB = 128
HALF = 64


def _tc_block(xb_ref, rows_ref, w1t_ref, w21t_ref,
              ndw_ref, thr_ref, y_ref, z_ref, *, k):
    xb = xb_ref[0]
    rows = rows_ref[0]
    r = rows.shape[0]

    gt = lax.dot_general(xb, rows, (((1,), (1,)), ((), ())),
                         preferred_element_type=jnp.float32)
    xx_all = jnp.sum(xb * xb, axis=1)
    xx_rows = jnp.sum(rows * rows, axis=1)
    ndt = 2.0 * gt - xx_all[:, None] - xx_rows[None, :]
    ndw_ref[0, 0] = ndt

    ibits = lax.bitcast_convert_type(ndt, jnp.int32)
    key = jnp.where(ibits < 0, ibits ^ jnp.int32(0x7FFFFFFF), ibits)

    def bs_body(_, carry):
        lo, hi = carry
        mid = (lo >> 1) + (hi >> 1) + (lo & hi & 1)
        cnt = jnp.sum((key >= mid[None, :]).astype(jnp.int32), axis=0)
        ge = cnt >= k
        return jnp.where(ge, mid, lo), jnp.where(ge, hi, mid)

    lo0 = jnp.full((r,), np.int32(-2**31), jnp.int32)
    hi0 = jnp.full((r,), np.int32(2**31 - 1), jnp.int32)
    lo, _ = lax.fori_loop(0, 32, bs_body, (lo0, hi0))
    thr_bits = jnp.where(lo < 0, lo ^ jnp.int32(0x7FFFFFFF), lo)
    thr_ref[0, 0] = lax.bitcast_convert_type(thr_bits, jnp.float32)

    y_ref[0] = jnp.dot(rows, w1t_ref[...], preferred_element_type=jnp.float32)
    z_ref[0] = jnp.dot(rows, w21t_ref[...], preferred_element_type=jnp.float32)


def _tc_stage(x_t, w1t, w21t, *, rblk):
    b, n, c = x_t.shape
    o = w21t.shape[1]
    nblk = n // rblk
    grid = (b, nblk)
    return pl.pallas_call(
        functools.partial(_tc_block, k=K),
        grid=grid,
        in_specs=[
            pl.BlockSpec((1, n, c), lambda bi, ri: (bi, 0, 0)),
            pl.BlockSpec((1, rblk, c), lambda bi, ri: (bi, ri, 0)),
            pl.BlockSpec((c, 2 * o), lambda bi, ri: (0, 0)),
            pl.BlockSpec((c, o), lambda bi, ri: (0, 0)),
        ],
        out_specs=[
            pl.BlockSpec((1, 1, n, rblk), lambda bi, ri: (bi, ri, 0, 0)),
            pl.BlockSpec((1, 1, rblk),
                         lambda bi, ri, _nb=nblk: (bi * _nb + ri, 0, 0)),
            pl.BlockSpec((1, rblk, 2 * o), lambda bi, ri: (bi, ri, 0)),
            pl.BlockSpec((1, rblk, o), lambda bi, ri: (bi, ri, 0)),
        ],
        out_shape=[
            jax.ShapeDtypeStruct((b, nblk, n, rblk), jnp.float32),
            jax.ShapeDtypeStruct((b * nblk, 1, rblk), jnp.float32),
            jax.ShapeDtypeStruct((b, n, 2 * o), jnp.float32),
            jax.ShapeDtypeStruct((b, n, o), jnp.float32),
        ],
    )(x_t, x_t, w1t, w21t)


def _sc_stage(ndw, thr_flat, y_flat, z_flat, *, n, o):
    total = thr_flat.shape[0]
    rpw = total // NW
    nblk = ndw.shape[1]
    rblk = n // nblk
    oq = o // LANES
    nslab = n // SLAB
    nhalf = rpw // HALF
    nu = nslab * nhalf
    grp = HALF // LANES
    qrows = HALF // 2

    mesh = plsc.VectorSubcoreMesh(core_axis_name="c", subcore_axis_name="s")

    @functools.partial(
        pl.kernel,
        out_type=jax.ShapeDtypeStruct((total, o), jnp.float32),
        mesh=mesh,
        compiler_params=pltpu.CompilerParams(needs_layout_passes=False),
        scratch_types=[
            pltpu.VMEM((SLAB, 128), jnp.float32),
            pltpu.VMEM((SLAB, 128), jnp.float32),
            pltpu.VMEM((rpw,), jnp.float32),
            pltpu.VMEM((HALF * NG,), jnp.int32),
            pltpu.VMEM((4 * NG, 2 * o), jnp.float32),
            pltpu.VMEM((4 * NG, 2 * o), jnp.float32),
            pltpu.VMEM((HALF // 2, o), jnp.float32),
            pltpu.VMEM((HALF // LANES, LANES), jnp.int32),
            pltpu.SemaphoreType.DMA,
            pltpu.SemaphoreType.DMA,
            pltpu.SemaphoreType.DMA,
            pltpu.SemaphoreType.DMA,
        ],
    )
    def sc_kernel(ndw_hbm, thr_hbm, y_hbm, z_hbm, out_hbm,
                  buf_a, buf_b, thr_v, idx_v, gat_a, gat_b, out_v, cnt_v,
                  sem_a, sem_b, sem_ga, sem_gb):
        wid = lax.axis_index("s") * NC + lax.axis_index("c")
        base = wid * rpw
        ri = base // rblk
        coff = base % rblk
        cal = (coff // 128) * 128
        boff = coff % 128
        ybase = 0

        pltpu.sync_copy(thr_hbm.at[pl.ds(base, rpw)], thr_v)

        iota = lax.broadcasted_iota(jnp.int32, (LANES,), 0)
        ngv = jnp.full((LANES,), NG, jnp.int32)
        zeros16 = jnp.zeros((LANES,), jnp.int32)

        def slab_copy(u, buf, sem):
            return pltpu.make_async_copy(
                ndw_hbm.at[0, ri,
                           pl.ds((u % nslab) * SLAB, SLAB),
                           pl.ds(cal, 128)],
                buf, sem)

        def process_slab(u, buf):
            jb = ybase + (u % nslab) * SLAB
            h0 = (u // nslab) * HALF
            for kg in range(grp):
                thr16 = thr_v[pl.ds(h0 + kg * LANES, LANES)]
                posb = (iota + kg * LANES) * NG

                def colb(c, cnt):
                    v = buf[c, pl.ds(boff + kg * LANES, LANES)]
                    msk = v >= thr16
                    ok = msk & (cnt < ngv)
                    val = jnp.full((LANES,), jb + c, jnp.int32)
                    plsc.store_scatter(idx_v, [posb + cnt], val, mask=ok)
                    return cnt + ok.astype(jnp.int32)

                cnt1 = lax.fori_loop(0, SLAB, colb,
                                     cnt_v[kg, pl.ds(0, LANES)], unroll=4)
                cnt_v[kg, pl.ds(0, LANES)] = cnt1

        def uloop(u, _):
            @pl.when(u % 2 == 0)
            def _():
                @pl.when(u + 1 < nu)
                def _():
                    slab_copy(u + 1, buf_b, sem_b).start()
                slab_copy(u, buf_a, sem_a).wait()
                process_slab(u, buf_a)

            @pl.when(u % 2 == 1)
            def _():
                @pl.when(u + 1 < nu)
                def _():
                    slab_copy(u + 1, buf_a, sem_a).start()
                slab_copy(u, buf_b, sem_b).wait()
                process_slab(u, buf_b)
            return 0

        def half_init(h):
            for kg in range(grp):
                cnt_v[kg, pl.ds(0, LANES)] = zeros16
                selfv = base + h * HALF + kg * LANES + iota
                posb = (iota + kg * LANES) * NG
                for sl in range(NG):
                    plsc.store_scatter(idx_v, [posb + sl], selfv)

        def gat_copy(i, gat, sem):
            return pltpu.make_async_copy(
                y_hbm.at[idx_v.at[pl.ds(i * NG, 4 * NG)]], gat, sem)

        def max_phase(jj, i_out, gat):
            acc = [gat[jj * NG, pl.ds(q * LANES, LANES)] for q in range(oq)]
            for j in range(1, NG):
                for q in range(oq):
                    acc[q] = jnp.maximum(
                        acc[q], gat[jj * NG + j, pl.ds(q * LANES, LANES)])
            for q in range(oq):
                res = acc[q] + out_v[i_out, pl.ds(q * LANES, LANES)]
                out_v[i_out, pl.ds(q * LANES, LANES)] = jnp.maximum(
                    res, NEG_SLOPE * res)

        def gather_half(h):
            qrows_ = HALF // 2
            for q2 in range(2):
                r0 = h * HALF + q2 * qrows_
                pltpu.sync_copy(z_hbm.at[pl.ds(base + r0, qrows_)], out_v)
                gat_copy(q2 * qrows_, gat_a, sem_ga).start()

                def pair_loop(tt, _):
                    il = 8 * tt
                    i0 = q2 * qrows_ + il
                    gat_copy(i0 + 4, gat_b, sem_gb).start()
                    gat_copy(i0, gat_a, sem_ga).wait()
                    for jj in range(4):
                        max_phase(jj, il + jj, gat_a)

                    @pl.when(il + 8 < qrows_)
                    def _():
                        gat_copy(i0 + 8, gat_a, sem_ga).start()
                    gat_copy(i0 + 4, gat_b, sem_gb).wait()
                    for jj in range(4):
                        max_phase(jj, il + 4 + jj, gat_b)
                    return 0

                lax.fori_loop(0, qrows_ // 8, pair_loop, 0)
                pltpu.sync_copy(out_v,
                                out_hbm.at[pl.ds(base + r0, qrows_)])

        slab_copy(0, buf_a, sem_a).start()
        for h in range(nhalf):
            half_init(h)
            lax.fori_loop(h * nslab, (h + 1) * nslab, uloop, 0)
            gather_half(h)

    return sc_kernel(ndw, thr_flat, y_flat, z_flat)


@jax.jit
def kernel(x, W):
    b, c, n = x.shape
    o = W.shape[0]
    x_t = jnp.transpose(x, (0, 2, 1))
    w1 = W[:, :c]
    w2 = W[:, c:]
    w1t = jnp.transpose(w1)
    w21t = jnp.transpose(w2 - w1)
    w1t_pad = jnp.pad(w1t, ((0, 0), (0, o)))
    outs = []
    for bb in range(b):
        ndw, thr, y, z = _tc_stage(x_t[bb:bb + 1], w1t_pad, w21t, rblk=256)
        outs.append(_sc_stage(ndw, thr.reshape(n),
                              y.reshape(n, 2 * o), z.reshape(n, o),
                              n=n, o=o))
    return jnp.transpose(jnp.stack(outs), (0, 2, 1))
